# baseline (speedup 1.0000x reference)
import functools

import jax
import jax.numpy as jnp
from jax.experimental import pallas as pl
from jax.experimental.pallas import tpu as pltpu


def _proj_kernel(x_ref, w_ref, h_ref):
    h_ref[...] = jnp.dot(
        x_ref[...].astype(jnp.bfloat16),
        w_ref[...].astype(jnp.bfloat16),
        preferred_element_type=jnp.float32,
    ).astype(jnp.bfloat16)


def _agg_kernel(adj_ref, h_ref, b_ref, out_ref):
    a = adj_ref[...].astype(jnp.bfloat16)
    acc = jnp.dot(a, h_ref[...], preferred_element_type=jnp.float32)
    out_ref[...] = acc + b_ref[...]


def kernel(input, adj, W, b):
    m, kin = input.shape
    kout = W.shape[1]
    n = adj.shape[1]

    bm_p = 2000 if m % 2000 == 0 else m
    h = pl.pallas_call(
        _proj_kernel,
        grid=(m // bm_p,),
        in_specs=[
            pl.BlockSpec((bm_p, kin), lambda i: (i, 0)),
            pl.BlockSpec((kin, kout), lambda i: (0, 0)),
        ],
        out_specs=pl.BlockSpec((bm_p, kout), lambda i: (i, 0)),
        out_shape=jax.ShapeDtypeStruct((m, kout), jnp.bfloat16),
        compiler_params=pltpu.CompilerParams(
            dimension_semantics=("parallel",),
        ),
    )(input, W)

    bm = 200 if m % 200 == 0 else m
    b2 = b.reshape(1, kout)
    out = pl.pallas_call(
        _agg_kernel,
        grid=(m // bm,),
        in_specs=[
            pl.BlockSpec((bm, n), lambda i: (i, 0)),
            pl.BlockSpec((n, kout), lambda i: (0, 0)),
            pl.BlockSpec((1, kout), lambda i: (0, 0)),
        ],
        out_specs=pl.BlockSpec((bm, kout), lambda i: (i, 0)),
        out_shape=jax.ShapeDtypeStruct((m, kout), jnp.float32),
        compiler_params=pltpu.CompilerParams(
            dimension_semantics=("parallel",),
        ),
    )(adj, h, b2)
    return out
